# Initial kernel scaffold; baseline (speedup 1.0000x reference)
import jax
import jax.numpy as jnp
from jax.experimental import pallas as pl


def kernel(input_ids, embed_table):
    raise NotImplementedError("write your pallas kernel here")


# speedup vs baseline: 1.6358x; 1.6358x over previous
import functools

import jax
import jax.numpy as jnp
from jax import lax
from jax.experimental import pallas as pl
from jax.experimental.pallas import tpu as pltpu
from jax.experimental.pallas import tpu_sc as plsc

_NC = 2
_NS = 16
_NW = _NC * _NS
_K = 32


@functools.lru_cache(maxsize=None)
def _gather_call(n, d, nch):
    mesh = plsc.VectorSubcoreMesh(core_axis_name="c", subcore_axis_name="s")

    @functools.partial(
        pl.kernel,
        mesh=mesh,
        out_type=jax.ShapeDtypeStruct((n, d), jnp.float32),
        scratch_types=[
            pltpu.VMEM((nch, _K), jnp.int32),
            pltpu.VMEM((_K, d), jnp.float32),
            pltpu.VMEM((_K, d), jnp.float32),
            pltpu.SemaphoreType.DMA,
            pltpu.SemaphoreType.DMA,
            pltpu.SemaphoreType.DMA,
            pltpu.SemaphoreType.DMA,
        ],
    )
    def grab(ids_hbm, table_hbm, out_hbm, idx_v, rows0, rows1, g0, g1, o0, o1):
        wid = lax.axis_index("s") * _NC + lax.axis_index("c")
        base = wid * (nch * _K)
        pltpu.sync_copy(ids_hbm.at[wid], idx_v)
        bufs = (rows0, rows1)
        gsems = (g0, g1)
        osems = (o0, o1)
        ghandles = [None, None]
        ohandles = [None, None]
        ghandles[0] = pltpu.async_copy(table_hbm.at[idx_v.at[0]], bufs[0], gsems[0])
        for ch in range(nch):
            b = ch & 1
            nb = 1 - b
            if ch + 1 < nch:
                if ohandles[nb] is not None:
                    ohandles[nb].wait()
                ghandles[nb] = pltpu.async_copy(
                    table_hbm.at[idx_v.at[ch + 1]], bufs[nb], gsems[nb])
            ghandles[b].wait()
            ohandles[b] = pltpu.async_copy(
                bufs[b], out_hbm.at[pl.ds(base + ch * _K, _K)], osems[b])
        for h in ohandles:
            if h is not None:
                h.wait()

    return grab


def kernel(input_ids, embed_table):
    b, s = input_ids.shape
    v, d = embed_table.shape
    n = b * s
    nch = n // (_NW * _K)
    ids = input_ids.reshape(_NW, nch, _K).astype(jnp.int32)
    out = _gather_call(n, d, nch)(ids, embed_table)
    return out.reshape(b, s, d)


# speedup vs baseline: 1.6633x; 1.0168x over previous
import functools

import jax
import jax.numpy as jnp
from jax import lax
from jax.experimental import pallas as pl
from jax.experimental.pallas import tpu as pltpu
from jax.experimental.pallas import tpu_sc as plsc

_NC = 2
_NS = 16
_NW = _NC * _NS
_DEPTH = 2
_K = 56


@functools.lru_cache(maxsize=None)
def _gather_call(n, d):
    per_w = n // _NW
    sizes = [_K] * (per_w // _K)
    if per_w % _K:
        sizes.append(per_w % _K)
    offs = [sum(sizes[:i]) for i in range(len(sizes))]
    nch = len(sizes)
    mesh = plsc.VectorSubcoreMesh(core_axis_name="c", subcore_axis_name="s")

    @functools.partial(
        pl.kernel,
        mesh=mesh,
        out_type=jax.ShapeDtypeStruct((n, d), jnp.float32),
        scratch_types=[pltpu.VMEM((per_w,), jnp.int32)]
        + [pltpu.VMEM((_K, d), jnp.float32)] * _DEPTH
        + [pltpu.SemaphoreType.DMA] * (2 * _DEPTH),
    )
    def grab(ids_hbm, table_hbm, out_hbm, idx_v, *rest):
        bufs = rest[:_DEPTH]
        gsems = rest[_DEPTH:2 * _DEPTH]
        osems = rest[2 * _DEPTH:]
        wid = lax.axis_index("s") * _NC + lax.axis_index("c")
        base = wid * per_w
        pltpu.sync_copy(ids_hbm.at[wid], idx_v)

        def gather(ch, b):
            sz = sizes[ch]
            dst = bufs[b] if sz == _K else bufs[b].at[pl.ds(0, sz)]
            return pltpu.async_copy(
                table_hbm.at[idx_v.at[pl.ds(offs[ch], sz)]], dst, gsems[b])

        ghandles = [None] * _DEPTH
        ohandles = [None] * _DEPTH
        for ch in range(min(_DEPTH, nch)):
            ghandles[ch] = gather(ch, ch)
        for ch in range(nch):
            b = ch % _DEPTH
            sz = sizes[ch]
            ghandles[b].wait()
            src = bufs[b] if sz == _K else bufs[b].at[pl.ds(0, sz)]
            ohandles[b] = pltpu.async_copy(
                src, out_hbm.at[pl.ds(base + offs[ch], sz)], osems[b])
            nxt = ch + _DEPTH
            if nxt < nch:
                ohandles[b].wait()
                ghandles[b] = gather(nxt, b)
        for h in ohandles:
            if h is not None:
                h.wait()

    return grab


def kernel(input_ids, embed_table):
    b, s = input_ids.shape
    v, d = embed_table.shape
    n = b * s
    ids = input_ids.reshape(_NW, n // _NW).astype(jnp.int32)
    out = _gather_call(n, d)(ids, embed_table)
    return out.reshape(b, s, d)
